# Initial kernel scaffold; baseline (speedup 1.0000x reference)
import jax
import jax.numpy as jnp
from jax.experimental import pallas as pl


def kernel(input, centroid):
    raise NotImplementedError("write your pallas kernel here")


# speedup vs baseline: 1.1586x; 1.1586x over previous
import jax
import jax.numpy as jnp
from jax.experimental import pallas as pl

_NUM_CENTROID = 8
_FEAT = 128 * 768
_ROWS = 256
_ROW_BLK = 32


def _body(x_ref, c_ref, o_ref):
    x = x_ref[...]
    c = c_ref[...]
    dots = jax.lax.dot_general(
        x, c, (((1,), (1,)), ((), ())),
        preferred_element_type=jnp.float32)
    c2 = jnp.sum(c * c, axis=1)
    score = c2[None, :] - 2.0 * dots
    o_ref[...] = jnp.argmin(score, axis=1).astype(jnp.int32)[:, None]


def kernel(input, centroid):
    x = input.reshape(_ROWS, _FEAT)
    grid = (_ROWS // _ROW_BLK,)
    out = pl.pallas_call(
        _body,
        grid=grid,
        in_specs=[
            pl.BlockSpec((_ROW_BLK, _FEAT), lambda i: (i, 0)),
            pl.BlockSpec((_NUM_CENTROID, _FEAT), lambda i: (0, 0)),
        ],
        out_specs=pl.BlockSpec((_ROW_BLK, 1), lambda i: (i, 0)),
        out_shape=jax.ShapeDtypeStruct((_ROWS, 1), jnp.int32),
    )(x, centroid)
    return out.reshape(_ROWS)
